# baseline (speedup 1.0000x reference)
import jax
import jax.numpy as jnp
from jax import lax
from jax.experimental import pallas as pl
from jax.experimental.pallas import tpu as pltpu
from jax.experimental.pallas import tpu_sc as plsc

N = 100000
D = 32
E = 1600000
L = 65536
NLAYER = 3
ALPHA = 1.0 / (NLAYER + 1)

NC = 2
NS = 16
LANES = 16
K = 128
GSZ = 8
NBUF = 4

HALF = N // NC
DUMP = HALF
ACC_ROWS = 50176
EPAD_CHUNKS = 12800
EPAD = EPAD_CHUNKS * K
NGROUPS = EPAD_CHUNKS // GSZ
GPT = NGROUPS // NS
NFULL = HALF // K
NREM = HALF - NFULL * K
ZCHUNKS = ACC_ROWS // K
DEGW = 8
PART_TILE = NFULL % NS

_F32 = jnp.float32
_I32 = jnp.int32


def _ntrips(limit, t):
    return (limit - t + NS - 1) // NS


def _fill_const(ref, value):
    v = jnp.full((LANES,), value, dtype=_F32)

    def body(r, _):
        ref[r, pl.ds(0, LANES)] = v
        ref[r, pl.ds(LANES, LANES)] = v
        return _

    lax.fori_loop(0, K, body, None)


def _zero_acc(t, acc_sh, zbuf):
    def body(i, _):
        zch = t + NS * i
        pltpu.sync_copy(zbuf, acc_sh.at[pl.ds(zch * K, K)])
        return _

    lax.fori_loop(0, _ntrips(ZCHUNKS, t), body, None)


def _stage_group(gid, base_node, col2_hbm, row2_hbm, cidx_g, ridx_g, lidx_g):
    cb = gid * GSZ
    pltpu.sync_copy(col2_hbm.at[pl.ds(cb, GSZ)], cidx_g)
    if row2_hbm is not None:
        pltpu.sync_copy(row2_hbm.at[pl.ds(cb, GSZ)], ridx_g)
    dump = jnp.full((LANES,), DUMP, dtype=_I32)

    def jbody(j, _):
        for v in range(K // LANES):
            sl = pl.ds(v * LANES, LANES)
            lc = cidx_g[j, sl] - base_node
            ok = (lc >= 0) & (lc < HALF)
            lidx_g[j, sl] = jnp.where(ok, lc, dump)
        return _

    lax.fori_loop(0, GSZ, jbody, None)


def _deg_body(col2_hbm, ones_hbm, zeros_hbm, deg_hbm,
              cidx_g0, cidx_g1, lidx_g0, lidx_g1, ones_b, zeros_b,
              ssem0, ssem1, acc_sh):
    c = lax.axis_index("c")
    t = lax.axis_index("s")
    pltpu.sync_copy(ones_hbm, ones_b)
    pltpu.sync_copy(zeros_hbm, zeros_b)
    _zero_acc(t, acc_sh, zeros_b)
    plsc.subcore_barrier()

    base_node = c * HALF
    cidx_g = (cidx_g0, cidx_g1)
    lidx_g = (lidx_g0, lidx_g1)
    ssem = (ssem0, ssem1)

    def drain(sl):
        for _ in range(GSZ):
            pltpu.make_async_copy(ones_b, acc_sh.at[lidx_g[sl].at[0]],
                                  ssem[sl]).wait()

    def group(u, sl):
        jj = 2 * u + sl
        gid = t + NS * jj
        _stage_group(gid, base_node, col2_hbm, None, cidx_g[sl], None,
                     lidx_g[sl])
        for j in range(GSZ):
            pltpu.async_copy(ones_b, acc_sh.at[lidx_g[sl].at[j]], ssem[sl],
                             add=True)

    def body(u, _):
        @pl.when(u > 0)
        def _():
            drain(0)

        group(u, 0)

        @pl.when(u > 0)
        def _():
            drain(1)

        group(u, 1)
        return _

    lax.fori_loop(0, GPT // 2, body, None)
    drain(0)
    drain(1)
    plsc.subcore_barrier()

    def node_chunk(nch, nrows):
        lbase = nch * K
        pltpu.sync_copy(acc_sh.at[pl.ds(lbase, nrows)],
                        deg_hbm.at[pl.ds(base_node + lbase, nrows)])

    def nbody(i, _):
        node_chunk(t + NS * i, K)
        return _

    lax.fori_loop(0, _ntrips(NFULL, t), nbody, None)

    @pl.when(t == PART_TILE)
    def _():
        node_chunk(NFULL, NREM)


def _scale_body(deg_ref, x_ref, s_ref, y_ref, o_ref):
    d = jnp.broadcast_to(deg_ref[...][:, :1], (deg_ref.shape[0], D))
    s = jnp.where(d > 0.5, lax.rsqrt(jnp.maximum(d, 1e-12)), 0.0)
    x = x_ref[...]
    s_ref[...] = s
    y_ref[...] = s * x
    o_ref[...] = ALPHA * x


def _layer_body(row2_hbm, col2_hbm, y_hbm, s_hbm, out_hbm,
                y_out, out_out,
                cidx_g0, cidx_g1, lidx_g0, lidx_g1, ridx_g0, ridx_g1,
                rows0, rows1, rows2, rows3,
                gsem0, gsem1, gsem2, gsem3,
                ssem0, ssem1, ssem2, ssem3, acc_sh):
    c = lax.axis_index("c")
    t = lax.axis_index("s")
    _fill_const(rows3, 0.0)
    _zero_acc(t, acc_sh, rows3)
    plsc.subcore_barrier()

    acc_t, s_t, o_t, y_t = rows0, rows1, rows2, rows3

    base_node = c * HALF
    cidx_g = (cidx_g0, cidx_g1)
    lidx_g = (lidx_g0, lidx_g1)
    ridx_g = (ridx_g0, ridx_g1)
    rows = (rows0, rows1, rows2, rows3)
    gsem = (gsem0, gsem1, gsem2, gsem3)
    ssem = (ssem0, ssem1, ssem2, ssem3)

    def swait(q, sl):
        pltpu.make_async_copy(rows[q], acc_sh.at[lidx_g[sl].at[0]],
                              ssem[q]).wait()

    def group(u, sl):
        jj = 2 * u + sl
        gid = t + NS * jj
        _stage_group(gid, base_node, col2_hbm, row2_hbm, cidx_g[sl],
                     ridx_g[sl], lidx_g[sl])
        if sl == 0:
            @pl.when(u > 0)
            def _():
                drain_tail(1 - sl)
        else:
            drain_tail(1 - sl)
        g = [pltpu.async_copy(y_hbm.at[ridx_g[sl].at[q]], rows[q], gsem[q])
             for q in range(NBUF)]
        for q in range(NBUF):
            g[q].wait()
            pltpu.async_copy(rows[q], acc_sh.at[lidx_g[sl].at[q]], ssem[q],
                             add=True)
        g2 = []
        for q in range(NBUF):
            swait(q, sl)
            g2.append(pltpu.async_copy(y_hbm.at[ridx_g[sl].at[NBUF + q]],
                                       rows[q], gsem[q]))
        for q in range(NBUF):
            g2[q].wait()
            pltpu.async_copy(rows[q], acc_sh.at[lidx_g[sl].at[NBUF + q]],
                             ssem[q], add=True)

    def drain_tail(sl):
        for q in range(NBUF):
            swait(q, sl)

    def body(u, _):
        group(u, 0)
        group(u, 1)
        return _

    lax.fori_loop(0, GPT // 2, body, None)
    drain_tail(1)
    plsc.subcore_barrier()

    def node_chunk(nch, nrows):
        lbase = nch * K
        gbase = base_node + lbase
        pltpu.sync_copy(acc_sh.at[pl.ds(lbase, nrows)],
                        acc_t.at[pl.ds(0, nrows)])
        pltpu.sync_copy(s_hbm.at[pl.ds(gbase, nrows)], s_t.at[pl.ds(0, nrows)])
        pltpu.sync_copy(out_hbm.at[pl.ds(gbase, nrows)],
                        o_t.at[pl.ds(0, nrows)])

        def row_body(r, _):
            for h in range(2):
                sl = pl.ds(h * LANES, LANES)
                sv = s_t[r, sl]
                sa = sv * acc_t[r, sl]
                y_t[r, sl] = sv * sa
                o_t[r, sl] = o_t[r, sl] + ALPHA * sa
            return _

        lax.fori_loop(0, nrows, row_body, None)
        pltpu.sync_copy(y_t.at[pl.ds(0, nrows)], y_out.at[pl.ds(gbase, nrows)])
        pltpu.sync_copy(o_t.at[pl.ds(0, nrows)],
                        out_out.at[pl.ds(gbase, nrows)])

    def nbody(i, _):
        node_chunk(t + NS * i, K)
        return _

    lax.fori_loop(0, _ntrips(NFULL, t), nbody, None)

    @pl.when(t == PART_TILE)
    def _():
        node_chunk(NFULL, NREM)


def _rank_body(a_hbm, b_hbm, out_hbm, rank_hbm,
               aidx, bidx, ra, rb, rk, gsem):
    c = lax.axis_index("c")
    t = lax.axis_index("s")
    w = t * NC + c
    chunks_per_w = L // K // (NC * NS)

    for i in range(chunks_per_w):
        base = (w * chunks_per_w + i) * K
        pltpu.sync_copy(a_hbm.at[pl.ds(base, K)], aidx)
        pltpu.sync_copy(b_hbm.at[pl.ds(base, K)], bidx)
        pltpu.async_copy(out_hbm.at[aidx], ra, gsem).wait()
        pltpu.async_copy(out_hbm.at[bidx], rb, gsem).wait()

        lane_id = lax.iota(_I32, LANES)

        def group_body(g, _):
            res = jnp.zeros((LANES,), dtype=_F32)
            for j in range(LANES):
                p = g * LANES + j
                pr = (ra[p, pl.ds(0, LANES)] * rb[p, pl.ds(0, LANES)]
                      + ra[p, pl.ds(LANES, LANES)]
                      * rb[p, pl.ds(LANES, LANES)])
                res = jnp.where(lane_id == j, jnp.sum(pr), res)
            rk[pl.ds(g * LANES, LANES)] = res
            return _

        lax.fori_loop(0, K // LANES, group_body, None)
        pltpu.sync_copy(rk, rank_hbm.at[pl.ds(base, K)])


_MESH = plsc.VectorSubcoreMesh(core_axis_name="c", subcore_axis_name="s")
_SC_PARAMS = pltpu.CompilerParams(use_tc_tiling_on_sc=False,
                                  needs_layout_passes=False)

_deg = pl.kernel(
    _deg_body,
    out_type=jax.ShapeDtypeStruct((N, DEGW), _F32),
    mesh=_MESH,
    compiler_params=_SC_PARAMS,
    scratch_types=(
        pltpu.VMEM((GSZ, K), _I32),
        pltpu.VMEM((GSZ, K), _I32),
        pltpu.VMEM((GSZ, K), _I32),
        pltpu.VMEM((GSZ, K), _I32),
        pltpu.VMEM((K, DEGW), _F32),
        pltpu.VMEM((K, DEGW), _F32),
        pltpu.SemaphoreType.DMA,
        pltpu.SemaphoreType.DMA,
        pltpu.VMEM_SHARED((ACC_ROWS, DEGW), _F32),
    ),
)

_SCALE_BLK = 1000

_scale = pl.pallas_call(
    _scale_body,
    grid=(N // _SCALE_BLK,),
    in_specs=[
        pl.BlockSpec((_SCALE_BLK, DEGW), lambda i: (i, 0)),
        pl.BlockSpec((_SCALE_BLK, D), lambda i: (i, 0)),
    ],
    out_specs=[
        pl.BlockSpec((_SCALE_BLK, D), lambda i: (i, 0)),
        pl.BlockSpec((_SCALE_BLK, D), lambda i: (i, 0)),
        pl.BlockSpec((_SCALE_BLK, D), lambda i: (i, 0)),
    ],
    out_shape=(
        jax.ShapeDtypeStruct((N, D), _F32),
        jax.ShapeDtypeStruct((N, D), _F32),
        jax.ShapeDtypeStruct((N, D), _F32),
    ),
)

_layer = pl.kernel(
    _layer_body,
    out_type=(
        jax.ShapeDtypeStruct((N, D), _F32),
        jax.ShapeDtypeStruct((N, D), _F32),
    ),
    mesh=_MESH,
    compiler_params=_SC_PARAMS,
    scratch_types=(
        pltpu.VMEM((GSZ, K), _I32),
        pltpu.VMEM((GSZ, K), _I32),
        pltpu.VMEM((GSZ, K), _I32),
        pltpu.VMEM((GSZ, K), _I32),
        pltpu.VMEM((GSZ, K), _I32),
        pltpu.VMEM((GSZ, K), _I32),
        pltpu.VMEM((K, D), _F32),
        pltpu.VMEM((K, D), _F32),
        pltpu.VMEM((K, D), _F32),
        pltpu.VMEM((K, D), _F32),
        pltpu.SemaphoreType.DMA,
        pltpu.SemaphoreType.DMA,
        pltpu.SemaphoreType.DMA,
        pltpu.SemaphoreType.DMA,
        pltpu.SemaphoreType.DMA,
        pltpu.SemaphoreType.DMA,
        pltpu.SemaphoreType.DMA,
        pltpu.SemaphoreType.DMA,
        pltpu.VMEM_SHARED((ACC_ROWS, D), _F32),
    ),
)

_rank = pl.kernel(
    _rank_body,
    out_type=jax.ShapeDtypeStruct((L,), _F32),
    mesh=_MESH,
    compiler_params=_SC_PARAMS,
    scratch_types=(
        pltpu.VMEM((K,), _I32),
        pltpu.VMEM((K,), _I32),
        pltpu.VMEM((K, D), _F32),
        pltpu.VMEM((K, D), _F32),
        pltpu.VMEM((K,), _F32),
        pltpu.SemaphoreType.DMA,
    ),
)


def kernel(edge_index, edge_label_index, emb):
    row = edge_index[0]
    col = edge_index[1]
    pad = EPAD - E
    row2 = jnp.concatenate([row, jnp.zeros((pad,), _I32)]).reshape(
        EPAD_CHUNKS, K)
    col2 = jnp.concatenate([col, jnp.full((pad,), 2 * N, _I32)]).reshape(
        EPAD_CHUNKS, K)
    deg = _deg(col2, jnp.ones((K, DEGW), _F32), jnp.zeros((K, DEGW), _F32))
    s, y, out = _scale(deg, emb)
    for _ in range(NLAYER):
        y, out = _layer(row2, col2, y, s, out)
    return _rank(edge_label_index[0], edge_label_index[1], out)


# speedup vs baseline: 1.1908x; 1.1908x over previous
import jax
import jax.numpy as jnp
from jax import lax
from jax.experimental import pallas as pl
from jax.experimental.pallas import tpu as pltpu
from jax.experimental.pallas import tpu_sc as plsc

N = 100000
D = 32
E = 1600000
L = 65536
NLAYER = 3
ALPHA = 1.0 / (NLAYER + 1)

NC = 2
NS = 16
LANES = 16
K = 128

HALF = N // NC
DUMP = HALF
ACC_ROWS = 50176
EPAD_CHUNKS = 12800
EPAD = EPAD_CHUNKS * K
PTCH = EPAD_CHUNKS // (NC * NS)
NFIX = 208
RSLOTS = NC * NS * NFIX
NFULL = HALF // K
NREM = HALF - NFULL * K
ZCHUNKS = ACC_ROWS // K
DEGW = 8
PART_TILE = NFULL % NS

_F32 = jnp.float32
_I32 = jnp.int32


def _ntrips(limit, t):
    return (limit - t + NS - 1) // NS


def _fill_const(ref, value):
    v = jnp.full((LANES,), value, dtype=_F32)

    def body(r, _):
        ref[r, pl.ds(0, LANES)] = v
        ref[r, pl.ds(LANES, LANES)] = v
        return _

    lax.fori_loop(0, K, body, None)


def _zero_acc(t, acc_sh, zbuf):
    def body(i, _):
        zch = t + NS * i
        pltpu.sync_copy(zbuf, acc_sh.at[pl.ds(zch * K, K)])
        return _

    lax.fori_loop(0, _ntrips(ZCHUNKS, t), body, None)


def _part_body(row2_hbm, col2_hbm, prow_hbm, plcl_hbm,
               rbuf, cbuf, br0, bl0, br1, bl1, padr, padl):
    c = lax.axis_index("c")
    t = lax.axis_index("s")
    wid = t * NC + c
    cbase = wid * PTCH
    sbase = wid * NFIX
    dump = jnp.full((LANES,), DUMP, dtype=_I32)
    zero16 = jnp.zeros((LANES,), dtype=_I32)
    br = (br0, br1)
    bl = (bl0, bl1)

    def fbody(w, _):
        sl = pl.ds(w * LANES, LANES)
        padl[sl] = dump
        padr[sl] = zero16
        return _

    lax.fori_loop(0, K // LANES, fbody, None)

    def chunk_body(i, carry):
        fs = [carry[0], carry[2]]
        ns = [carry[1], carry[3]]
        ch = cbase + i
        pltpu.sync_copy(row2_hbm.at[ch], rbuf)
        pltpu.sync_copy(col2_hbm.at[ch], cbuf)
        for v in range(K // LANES):
            sl = pl.ds(v * LANES, LANES)
            cv = cbuf[sl]
            rv = rbuf[sl]
            for h in range(2):
                lc = cv - h * HALF
                ok = (lc >= 0) & (lc < HALF)
                plsc.store_compressed(br[h].at[pl.ds(fs[h], LANES)], rv, mask=ok)
                plsc.store_compressed(bl[h].at[pl.ds(fs[h], LANES)], lc, mask=ok)
                fs[h] = fs[h] + jnp.sum(ok.astype(_I32))
                need = fs[h] >= K
                slot = sbase + jnp.minimum(ns[h], NFIX - 1)

                @pl.when(need)
                def _(h=h, slot=slot):
                    pltpu.sync_copy(br[h].at[pl.ds(0, K)],
                                    prow_hbm.at[h, slot])
                    pltpu.sync_copy(bl[h].at[pl.ds(0, K)],
                                    plcl_hbm.at[h, slot])
                    tr = br[h][pl.ds(K, LANES)]
                    tl = bl[h][pl.ds(K, LANES)]
                    br[h][pl.ds(0, LANES)] = tr
                    bl[h][pl.ds(0, LANES)] = tl

                fs[h] = jnp.where(need, fs[h] - K, fs[h])
                ns[h] = jnp.where(need, ns[h] + 1, ns[h])
        return fs[0], ns[0], fs[1], ns[1]

    z = jnp.int32(0)
    f0, n0, f1, n1 = lax.fori_loop(0, PTCH, chunk_body, (z, z, z, z))

    for h, (f, n) in enumerate(((f0, n0), (f1, n1))):
        def sbody(w, _, h=h, f=f):
            sl = pl.ds(w * LANES, LANES)
            lane = lax.iota(_I32, LANES) + w * LANES
            keep = lane < f
            bl[h][sl] = jnp.where(keep, bl[h][sl], dump)
            br[h][sl] = jnp.where(keep, br[h][sl], zero16)
            return _

        lax.fori_loop(0, K // LANES, sbody, None)
        slot = sbase + jnp.minimum(n, NFIX - 1)

        @pl.when(f > 0)
        def _(h=h, slot=slot):
            pltpu.sync_copy(br[h].at[pl.ds(0, K)], prow_hbm.at[h, slot])
            pltpu.sync_copy(bl[h].at[pl.ds(0, K)], plcl_hbm.at[h, slot])

        nfin = jnp.where(f > 0, n + 1, n)

        def pbody(m, _, h=h):
            pltpu.sync_copy(padr, prow_hbm.at[h, sbase + m])
            pltpu.sync_copy(padl, plcl_hbm.at[h, sbase + m])
            return _

        lax.fori_loop(nfin, NFIX, pbody, None)


def _deg_body(plcl_hbm, ones_hbm, zeros_hbm, deg_hbm,
              lidx0, lidx1, ones_b, zeros_b, ssem0, ssem1, acc_sh):
    c = lax.axis_index("c")
    t = lax.axis_index("s")
    pltpu.sync_copy(ones_hbm, ones_b)
    pltpu.sync_copy(zeros_hbm, zeros_b)
    _zero_acc(t, acc_sh, zeros_b)
    plsc.subcore_barrier()

    base_node = c * HALF
    lidx = (lidx0, lidx1)
    ssem = (ssem0, ssem1)
    tbase = 2 * t * NFIX

    def swait(sl):
        pltpu.make_async_copy(ones_b, acc_sh.at[lidx[sl]], ssem[sl]).wait()

    def body(g, _):
        @pl.when(g > 0)
        def _():
            swait(0)
            swait(1)

        for sl in (0, 1):
            pltpu.sync_copy(plcl_hbm.at[c, tbase + 2 * g + sl], lidx[sl])
            pltpu.async_copy(ones_b, acc_sh.at[lidx[sl]], ssem[sl], add=True)
        return _

    lax.fori_loop(0, NFIX, body, None)
    swait(0)
    swait(1)
    plsc.subcore_barrier()

    def node_chunk(nch, nrows):
        lbase = nch * K
        pltpu.sync_copy(acc_sh.at[pl.ds(lbase, nrows)],
                        deg_hbm.at[pl.ds(base_node + lbase, nrows)])

    def nbody(i, _):
        node_chunk(t + NS * i, K)
        return _

    lax.fori_loop(0, _ntrips(NFULL, t), nbody, None)

    @pl.when(t == PART_TILE)
    def _():
        node_chunk(NFULL, NREM)


def _scale_body(deg_ref, x_ref, s_ref, y_ref, o_ref):
    d = jnp.broadcast_to(deg_ref[...][:, :1], (deg_ref.shape[0], D))
    s = jnp.where(d > 0.5, lax.rsqrt(jnp.maximum(d, 1e-12)), 0.0)
    x = x_ref[...]
    s_ref[...] = s
    y_ref[...] = s * x
    o_ref[...] = ALPHA * x


def _layer_body(prow_hbm, plcl_hbm, y_hbm, s_hbm, out_hbm,
                y_out, out_out,
                lidx0, lidx1, ridx0, ridx1,
                rows0, rows1, rows2, rows3,
                gsem0, gsem1, ssem0, ssem1, acc_sh):
    c = lax.axis_index("c")
    t = lax.axis_index("s")
    _fill_const(rows3, 0.0)
    _zero_acc(t, acc_sh, rows3)
    plsc.subcore_barrier()

    base_node = c * HALF
    lidx = (lidx0, lidx1)
    ridx = (ridx0, ridx1)
    rows = (rows0, rows1)
    gsem = (gsem0, gsem1)
    ssem = (ssem0, ssem1)
    tbase = 2 * t * NFIX

    def swait(sl):
        pltpu.make_async_copy(rows[sl], acc_sh.at[lidx[sl]], ssem[sl]).wait()

    def body(g, _):
        @pl.when(g > 0)
        def _():
            swait(0)
            swait(1)

        pltpu.sync_copy(prow_hbm.at[c, tbase + 2 * g], ridx[0])
        pltpu.sync_copy(plcl_hbm.at[c, tbase + 2 * g], lidx[0])
        d0 = pltpu.async_copy(y_hbm.at[ridx[0]], rows[0], gsem[0])
        pltpu.sync_copy(prow_hbm.at[c, tbase + 2 * g + 1], ridx[1])
        pltpu.sync_copy(plcl_hbm.at[c, tbase + 2 * g + 1], lidx[1])
        d1 = pltpu.async_copy(y_hbm.at[ridx[1]], rows[1], gsem[1])
        d0.wait()
        pltpu.async_copy(rows[0], acc_sh.at[lidx[0]], ssem[0], add=True)
        d1.wait()
        pltpu.async_copy(rows[1], acc_sh.at[lidx[1]], ssem[1], add=True)
        return _

    lax.fori_loop(0, NFIX, body, None)
    swait(0)
    swait(1)
    plsc.subcore_barrier()

    acc_t, s_t, o_t, y_t = rows0, rows1, rows2, rows3

    def node_chunk(nch, nrows):
        lbase = nch * K
        gbase = base_node + lbase
        pltpu.sync_copy(acc_sh.at[pl.ds(lbase, nrows)],
                        acc_t.at[pl.ds(0, nrows)])
        pltpu.sync_copy(s_hbm.at[pl.ds(gbase, nrows)], s_t.at[pl.ds(0, nrows)])
        pltpu.sync_copy(out_hbm.at[pl.ds(gbase, nrows)],
                        o_t.at[pl.ds(0, nrows)])

        def row_body(r, _):
            for h in range(2):
                sl = pl.ds(h * LANES, LANES)
                sv = s_t[r, sl]
                sa = sv * acc_t[r, sl]
                y_t[r, sl] = sv * sa
                o_t[r, sl] = o_t[r, sl] + ALPHA * sa
            return _

        lax.fori_loop(0, nrows, row_body, None)
        pltpu.sync_copy(y_t.at[pl.ds(0, nrows)], y_out.at[pl.ds(gbase, nrows)])
        pltpu.sync_copy(o_t.at[pl.ds(0, nrows)],
                        out_out.at[pl.ds(gbase, nrows)])

    def nbody(i, _):
        node_chunk(t + NS * i, K)
        return _

    lax.fori_loop(0, _ntrips(NFULL, t), nbody, None)

    @pl.when(t == PART_TILE)
    def _():
        node_chunk(NFULL, NREM)


def _rank_body(a_hbm, b_hbm, out_hbm, rank_hbm,
               aidx, bidx, ra, rb, rk, gsem):
    c = lax.axis_index("c")
    t = lax.axis_index("s")
    w = t * NC + c
    chunks_per_w = L // K // (NC * NS)

    for i in range(chunks_per_w):
        base = (w * chunks_per_w + i) * K
        pltpu.sync_copy(a_hbm.at[pl.ds(base, K)], aidx)
        pltpu.sync_copy(b_hbm.at[pl.ds(base, K)], bidx)
        pltpu.async_copy(out_hbm.at[aidx], ra, gsem).wait()
        pltpu.async_copy(out_hbm.at[bidx], rb, gsem).wait()

        lane_id = lax.iota(_I32, LANES)

        def group_body(g, _):
            res = jnp.zeros((LANES,), dtype=_F32)
            for j in range(LANES):
                p = g * LANES + j
                pr = (ra[p, pl.ds(0, LANES)] * rb[p, pl.ds(0, LANES)]
                      + ra[p, pl.ds(LANES, LANES)]
                      * rb[p, pl.ds(LANES, LANES)])
                res = jnp.where(lane_id == j, jnp.sum(pr), res)
            rk[pl.ds(g * LANES, LANES)] = res
            return _

        lax.fori_loop(0, K // LANES, group_body, None)
        pltpu.sync_copy(rk, rank_hbm.at[pl.ds(base, K)])


_MESH = plsc.VectorSubcoreMesh(core_axis_name="c", subcore_axis_name="s")
_SC_PARAMS = pltpu.CompilerParams(use_tc_tiling_on_sc=False,
                                  needs_layout_passes=False)

_part = pl.kernel(
    _part_body,
    out_type=(
        jax.ShapeDtypeStruct((NC, RSLOTS, K), _I32),
        jax.ShapeDtypeStruct((NC, RSLOTS, K), _I32),
    ),
    mesh=_MESH,
    compiler_params=_SC_PARAMS,
    scratch_types=(
        pltpu.VMEM((K,), _I32),
        pltpu.VMEM((K,), _I32),
        pltpu.VMEM((K + LANES,), _I32),
        pltpu.VMEM((K + LANES,), _I32),
        pltpu.VMEM((K + LANES,), _I32),
        pltpu.VMEM((K + LANES,), _I32),
        pltpu.VMEM((K,), _I32),
        pltpu.VMEM((K,), _I32),
    ),
)

_deg = pl.kernel(
    _deg_body,
    out_type=jax.ShapeDtypeStruct((N, DEGW), _F32),
    mesh=_MESH,
    compiler_params=_SC_PARAMS,
    scratch_types=(
        pltpu.VMEM((K,), _I32),
        pltpu.VMEM((K,), _I32),
        pltpu.VMEM((K, DEGW), _F32),
        pltpu.VMEM((K, DEGW), _F32),
        pltpu.SemaphoreType.DMA,
        pltpu.SemaphoreType.DMA,
        pltpu.VMEM_SHARED((ACC_ROWS, DEGW), _F32),
    ),
)

_SCALE_BLK = 1000

_scale = pl.pallas_call(
    _scale_body,
    grid=(N // _SCALE_BLK,),
    in_specs=[
        pl.BlockSpec((_SCALE_BLK, DEGW), lambda i: (i, 0)),
        pl.BlockSpec((_SCALE_BLK, D), lambda i: (i, 0)),
    ],
    out_specs=[
        pl.BlockSpec((_SCALE_BLK, D), lambda i: (i, 0)),
        pl.BlockSpec((_SCALE_BLK, D), lambda i: (i, 0)),
        pl.BlockSpec((_SCALE_BLK, D), lambda i: (i, 0)),
    ],
    out_shape=(
        jax.ShapeDtypeStruct((N, D), _F32),
        jax.ShapeDtypeStruct((N, D), _F32),
        jax.ShapeDtypeStruct((N, D), _F32),
    ),
)

_layer = pl.kernel(
    _layer_body,
    out_type=(
        jax.ShapeDtypeStruct((N, D), _F32),
        jax.ShapeDtypeStruct((N, D), _F32),
    ),
    mesh=_MESH,
    compiler_params=_SC_PARAMS,
    scratch_types=(
        pltpu.VMEM((K,), _I32),
        pltpu.VMEM((K,), _I32),
        pltpu.VMEM((K,), _I32),
        pltpu.VMEM((K,), _I32),
        pltpu.VMEM((K, D), _F32),
        pltpu.VMEM((K, D), _F32),
        pltpu.VMEM((K, D), _F32),
        pltpu.VMEM((K, D), _F32),
        pltpu.SemaphoreType.DMA,
        pltpu.SemaphoreType.DMA,
        pltpu.SemaphoreType.DMA,
        pltpu.SemaphoreType.DMA,
        pltpu.VMEM_SHARED((ACC_ROWS, D), _F32),
    ),
)

_rank = pl.kernel(
    _rank_body,
    out_type=jax.ShapeDtypeStruct((L,), _F32),
    mesh=_MESH,
    compiler_params=_SC_PARAMS,
    scratch_types=(
        pltpu.VMEM((K,), _I32),
        pltpu.VMEM((K,), _I32),
        pltpu.VMEM((K, D), _F32),
        pltpu.VMEM((K, D), _F32),
        pltpu.VMEM((K,), _F32),
        pltpu.SemaphoreType.DMA,
    ),
)


def kernel(edge_index, edge_label_index, emb):
    row = edge_index[0]
    col = edge_index[1]
    pad = EPAD - E
    row2 = jnp.concatenate([row, jnp.zeros((pad,), _I32)]).reshape(
        EPAD_CHUNKS, K)
    col2 = jnp.concatenate([col, jnp.full((pad,), 2 * N, _I32)]).reshape(
        EPAD_CHUNKS, K)
    prow, plcl = _part(row2, col2)
    deg = _deg(plcl, jnp.ones((K, DEGW), _F32), jnp.zeros((K, DEGW), _F32))
    s, y, out = _scale(deg, emb)
    for _ in range(NLAYER):
        y, out = _layer(prow, plcl, y, s, out)
    return _rank(edge_label_index[0], edge_label_index[1], out)
